# baseline (speedup 1.0000x reference)
import functools

import jax
import jax.numpy as jnp
from jax import lax
from jax.experimental import pallas as pl
from jax.experimental.pallas import tpu as pltpu
from jax.experimental.pallas import tpu_sc as plsc

EMB_DIM = 64
SCALE = 8.0
NUM_CORES = 2
NUM_SUBCORES = 16
NW = NUM_CORES * NUM_SUBCORES
CHUNK = 128
LANES = 16

_MESH = dict(core_axis_name="c", subcore_axis_name="s",
             num_cores=NUM_CORES, num_subcores=NUM_SUBCORES)


@functools.cache
def _make_transpose(V):
    n_full = V // CHUNK
    tail = V - n_full * CHUNK
    mesh = plsc.VectorSubcoreMesh(**_MESH)

    @functools.partial(
        pl.kernel,
        out_type=jax.ShapeDtypeStruct((V // 2, 2 * EMB_DIM), jnp.float32),
        mesh=mesh,
        scratch_types=[
            pltpu.VMEM((2, EMB_DIM, CHUNK), jnp.float32),
            pltpu.VMEM((2, CHUNK // 2, 2 * EMB_DIM), jnp.float32),
            pltpu.VMEM((EMB_DIM, tail or 1), jnp.float32),
            pltpu.VMEM(((tail or 2) // 2, 2 * EMB_DIM), jnp.float32),
            pltpu.SemaphoreType.DMA,
            pltpu.SemaphoreType.DMA,
        ],
        compiler_params=pltpu.CompilerParams(use_tc_tiling_on_sc=True,
                                             needs_layout_passes=False),
    )
    def transpose(tp_hbm, out_hbm, tin, tout, tin_t, tout_t, sem_i, sem_o):
        wid = lax.axis_index("s") * NUM_CORES + lax.axis_index("c")
        k_max = (n_full - wid + NW - 1) // NW

        def col_of(k):
            return pl.multiple_of((wid + k * NW) * CHUNK, CHUNK)

        def start_in(k, b):
            pltpu.async_copy(
                tp_hbm.at[:, pl.ds(col_of(k), CHUNK)], tin.at[b], sem_i)

        def wait_in(k, b):
            pltpu.make_async_copy(
                tp_hbm.at[:, pl.ds(col_of(k), CHUNK)], tin.at[b], sem_i
            ).wait()

        def start_out(k, b):
            pltpu.async_copy(
                tout.at[b],
                out_hbm.at[pl.ds(pl.multiple_of(col_of(k) // 2, CHUNK // 2),
                                 CHUNK // 2)],
                sem_o)

        def wait_out(k, b):
            pltpu.make_async_copy(
                tout.at[b],
                out_hbm.at[pl.ds(pl.multiple_of(col_of(k) // 2, CHUNK // 2),
                                 CHUNK // 2)],
                sem_o).wait()

        def compute(tin_ref, tout_ref, n_cols):
            def body(q, _):
                for h in range(2):
                    for j in range(EMB_DIM // LANES):
                        rows = jnp.arange(LANES, dtype=jnp.int32) + j * LANES
                        cols = jnp.full((LANES,), 2 * q + h, jnp.int32)
                        vals = plsc.load_gather(tin_ref, [rows, cols])
                        tout_ref[q, pl.ds(h * EMB_DIM + j * LANES, LANES)] = (
                            vals)
                return ()

            lax.fori_loop(0, n_cols // 2, body, ())

        @pl.when(k_max > 0)
        def _():
            start_in(0, 0)

            def step(k, _):
                b = lax.rem(k, 2)

                @pl.when(k + 1 < k_max)
                def _():
                    start_in(k + 1, 1 - b)

                wait_in(k, b)

                @pl.when(k >= 2)
                def _():
                    wait_out(k - 2, b)

                compute(tin.at[b], tout.at[b], CHUNK)
                start_out(k, b)
                return ()

            lax.fori_loop(0, k_max, step, ())

            @pl.when(k_max >= 2)
            def _():
                wait_out(k_max - 2, lax.rem(k_max, 2))

            wait_out(k_max - 1, lax.rem(k_max - 1, 2))

        if tail:
            @pl.when(wid == 0)
            def _():
                pltpu.sync_copy(
                    tp_hbm.at[:, pl.ds(n_full * CHUNK, tail)], tin_t)
                compute(tin_t, tout_t, tail)
                pltpu.sync_copy(
                    tout_t,
                    out_hbm.at[pl.ds(n_full * CHUNK // 2, tail // 2)])

    return transpose


@functools.cache
def _make_gather(B, V):
    assert B % (NW * CHUNK) == 0
    b_per_w = B // NW
    g_per_w = b_per_w // CHUNK
    mesh = plsc.VectorSubcoreMesh(**_MESH)

    @functools.partial(
        pl.kernel,
        out_type=jax.ShapeDtypeStruct((B, EMB_DIM), jnp.float32),
        mesh=mesh,
        scratch_types=[
            pltpu.VMEM((g_per_w, CHUNK), jnp.int32),
            pltpu.VMEM((CHUNK,), jnp.int32),
            pltpu.VMEM((CHUNK, 2 * EMB_DIM), jnp.float32),
            pltpu.VMEM((CHUNK, EMB_DIM), jnp.float32),
            pltpu.SemaphoreType.DMA,
        ],
        compiler_params=pltpu.CompilerParams(use_tc_tiling_on_sc=True,
                                             needs_layout_passes=False),
    )
    def gather(tok_hbm, table_hbm, out_hbm, tok_v, pk_v, rows_v, out_v, sem):
        wid = lax.axis_index("s") * NUM_CORES + lax.axis_index("c")
        base = wid * b_per_w
        pltpu.sync_copy(
            tok_hbm.at[pl.ds(pl.multiple_of(wid * g_per_w, 8), g_per_w)],
            tok_v)

        def chunk_body(g, _):
            def pk_body(u, _):
                sl = pl.ds(u * LANES, LANES)
                pk_v[sl] = tok_v[g, sl] >> 1
                return ()

            lax.fori_loop(0, CHUNK // LANES, pk_body, ())
            pltpu.async_copy(table_hbm.at[pk_v], rows_v, sem).wait()

            def scale_body(u, _):
                halves = (tok_v[g, pl.ds(u * LANES, LANES)] & 1) * EMB_DIM
                for i in range(LANES):
                    h = halves[i]
                    r = u * LANES + i
                    for j in range(EMB_DIM // LANES):
                        src = pl.ds(h + j * LANES, LANES)
                        out_v[r, pl.ds(j * LANES, LANES)] = (
                            rows_v[r, src] * SCALE)
                return ()

            lax.fori_loop(0, CHUNK // LANES, scale_body, ())
            pltpu.sync_copy(
                out_v,
                out_hbm.at[pl.ds(pl.multiple_of(base + g * CHUNK, CHUNK),
                                 CHUNK)])
            return ()

        lax.fori_loop(0, g_per_w, chunk_body, ())

    return gather


def kernel(tokens, table):
    B = tokens.size
    V = table.shape[0]
    toks = tokens.reshape(-1).astype(jnp.int32).reshape(-1, CHUNK)
    tphys = jnp.transpose(table)
    packed = _make_transpose(V)(tphys)
    out = _make_gather(B, V)(toks, packed)
    return out.reshape(tokens.shape + (EMB_DIM,))


# speedup vs baseline: 1.8172x; 1.8172x over previous
import functools

import jax
import jax.numpy as jnp
from jax import lax
from jax.experimental import pallas as pl
from jax.experimental.pallas import tpu as pltpu
from jax.experimental.pallas import tpu_sc as plsc

EMB_DIM = 64
SCALE = 8.0
NUM_CORES = 2
NUM_SUBCORES = 16
NW = NUM_CORES * NUM_SUBCORES
CHUNK = 128
LANES = 16

_MESH = dict(core_axis_name="c", subcore_axis_name="s",
             num_cores=NUM_CORES, num_subcores=NUM_SUBCORES)


@functools.cache
def _make_transpose(V):
    n_full = V // CHUNK
    tail = V - n_full * CHUNK
    mesh = plsc.VectorSubcoreMesh(**_MESH)

    @functools.partial(
        pl.kernel,
        out_type=jax.ShapeDtypeStruct((V // 2, 2 * EMB_DIM), jnp.float32),
        mesh=mesh,
        scratch_types=[
            pltpu.VMEM((2, EMB_DIM, CHUNK), jnp.float32),
            pltpu.VMEM((2, CHUNK // 2, 2 * EMB_DIM), jnp.float32),
            pltpu.VMEM((EMB_DIM, tail or 1), jnp.float32),
            pltpu.VMEM(((tail or 2) // 2, 2 * EMB_DIM), jnp.float32),
            pltpu.SemaphoreType.DMA,
            pltpu.SemaphoreType.DMA,
        ],
        compiler_params=pltpu.CompilerParams(use_tc_tiling_on_sc=True,
                                             needs_layout_passes=False),
    )
    def transpose(tp_hbm, out_hbm, tin, tout, tin_t, tout_t, sem_i, sem_o):
        wid = lax.axis_index("s") * NUM_CORES + lax.axis_index("c")
        k_max = (n_full - wid + NW - 1) // NW
        rows_tab = [jnp.arange(LANES, dtype=jnp.int32) + j * LANES
                    for j in range(EMB_DIM // LANES)]

        def col_of(k):
            return pl.multiple_of((wid + k * NW) * CHUNK, CHUNK)

        def start_in(k, b):
            pltpu.async_copy(
                tp_hbm.at[:, pl.ds(col_of(k), CHUNK)], tin.at[b], sem_i)

        def wait_in(k, b):
            pltpu.make_async_copy(
                tp_hbm.at[:, pl.ds(col_of(k), CHUNK)], tin.at[b], sem_i
            ).wait()

        def start_out(k, b):
            pltpu.async_copy(
                tout.at[b],
                out_hbm.at[pl.ds(pl.multiple_of(col_of(k) // 2, CHUNK // 2),
                                 CHUNK // 2)],
                sem_o)

        def wait_out(k, b):
            pltpu.make_async_copy(
                tout.at[b],
                out_hbm.at[pl.ds(pl.multiple_of(col_of(k) // 2, CHUNK // 2),
                                 CHUNK // 2)],
                sem_o).wait()

        def compute(tin_ref, tout_ref, n_cols):
            @plsc.parallel_loop(0, n_cols // 2, 1, unroll=4)
            def _(q):
                for h in range(2):
                    cols = jnp.full((LANES,), 2 * q + h, jnp.int32)
                    for j in range(EMB_DIM // LANES):
                        vals = plsc.load_gather(tin_ref, [rows_tab[j], cols])
                        tout_ref[q, pl.ds(h * EMB_DIM + j * LANES, LANES)] = (
                            vals)

        @pl.when(k_max > 0)
        def _():
            start_in(0, 0)

            def step(k, _):
                b = lax.rem(k, 2)

                @pl.when(k + 1 < k_max)
                def _():
                    start_in(k + 1, 1 - b)

                wait_in(k, b)

                @pl.when(k >= 2)
                def _():
                    wait_out(k - 2, b)

                compute(tin.at[b], tout.at[b], CHUNK)
                start_out(k, b)
                return ()

            lax.fori_loop(0, k_max, step, ())

            @pl.when(k_max >= 2)
            def _():
                wait_out(k_max - 2, lax.rem(k_max, 2))

            wait_out(k_max - 1, lax.rem(k_max - 1, 2))

        if tail:
            @pl.when(wid == 0)
            def _():
                pltpu.sync_copy(
                    tp_hbm.at[:, pl.ds(n_full * CHUNK, tail)], tin_t)
                compute(tin_t, tout_t, tail)
                pltpu.sync_copy(
                    tout_t,
                    out_hbm.at[pl.ds(n_full * CHUNK // 2, tail // 2)])

    return transpose


@functools.cache
def _make_gather(B, V):
    assert B % (NW * CHUNK) == 0
    b_per_w = B // NW
    g_per_w = b_per_w // CHUNK
    mesh = plsc.VectorSubcoreMesh(**_MESH)

    @functools.partial(
        pl.kernel,
        out_type=jax.ShapeDtypeStruct((B, EMB_DIM), jnp.float32),
        mesh=mesh,
        scratch_types=[
            pltpu.VMEM((g_per_w, CHUNK), jnp.int32),
            pltpu.VMEM((CHUNK, EMB_DIM), jnp.float32),
            pltpu.VMEM((CHUNK, EMB_DIM), jnp.float32),
            pltpu.SemaphoreType.DMA,
        ],
        compiler_params=pltpu.CompilerParams(use_tc_tiling_on_sc=False),
    )
    def gather(tok_hbm, table_hbm, out_hbm, tok_v, rows_v, out_v, sem):
        wid = lax.axis_index("s") * NUM_CORES + lax.axis_index("c")
        base = wid * b_per_w
        pltpu.sync_copy(tok_hbm.at[pl.ds(wid * g_per_w, g_per_w)], tok_v)

        def chunk_body(g, _):
            pltpu.async_copy(table_hbm.at[tok_v.at[g]], rows_v, sem).wait()

            def scale_body(r, _):
                for j in range(EMB_DIM // LANES):
                    sl = pl.ds(j * LANES, LANES)
                    out_v[r, sl] = rows_v[r, sl] * SCALE
                return ()

            lax.fori_loop(0, CHUNK, scale_body, ())
            pltpu.sync_copy(out_v, out_hbm.at[pl.ds(base + g * CHUNK, CHUNK)])
            return ()

        lax.fori_loop(0, g_per_w, chunk_body, ())

    return gather


def kernel(tokens, table):
    B = tokens.size
    V = table.shape[0]
    toks = tokens.reshape(-1).astype(jnp.int32).reshape(-1, CHUNK)
    tphys = jnp.transpose(table)
    packed = _make_transpose(V)(tphys)
    table_lin = packed.reshape(V, EMB_DIM)
    out = _make_gather(B, V)(toks, table_lin)
    return out.reshape(tokens.shape + (EMB_DIM,))
